# baseline (device time: 19891 ns/iter reference)
import jax
import jax.numpy as jnp
from jax import lax
from jax.experimental import pallas as pl
from jax.experimental.pallas import tpu as pltpu

N_DEV = 4

QCLIP = 5.0
INV_SCALE = 127.0 / QCLIP
DEQ_SCALE = QCLIP / 127.0


def kernel(x, w_mat):
    m_total, k_shard = x.shape
    k_total, n = w_mat.shape
    m_per = m_total // N_DEV

    def body(x_ref, w_hbm, out_ref,
             q_stage, q_comm, xbf, w_full,
             send_sems, recv_sems, w_sems):
        my = lax.axis_index("i")

        w_cps = []
        for h in range(2):
            cp = pltpu.make_async_copy(
                w_hbm.at[pl.ds(h * (k_total // 2), k_total // 2), :],
                w_full.at[pl.ds(h * (k_total // 2), k_total // 2), :],
                w_sems.at[h],
            )
            cp.start()
            w_cps.append(cp)

        barrier_sem = pltpu.get_barrier_semaphore()
        for d in range(1, N_DEV):
            peer = lax.rem(my + d, N_DEV)
            pl.semaphore_signal(
                barrier_sem, inc=1,
                device_id=(peer,), device_id_type=pl.DeviceIdType.MESH,
            )

        def quantize(d):
            peer = lax.rem(my + d, N_DEV)
            blk = x_ref[pl.ds(peer * m_per, m_per), :]
            q_stage[d] = jnp.clip(
                jnp.round(blk * INV_SCALE), -127.0, 127.0
            ).astype(jnp.int8)

        quantize(1)
        pl.semaphore_wait(barrier_sem, N_DEV - 1)

        sends = []
        for d in range(1, N_DEV):
            peer = lax.rem(my + d, N_DEV)
            rdma = pltpu.make_async_remote_copy(
                src_ref=q_stage.at[d],
                dst_ref=q_comm.at[my],
                send_sem=send_sems.at[d - 1],
                recv_sem=recv_sems.at[my],
                device_id=(peer,),
                device_id_type=pl.DeviceIdType.MESH,
            )
            rdma.start()
            sends.append(rdma)
            if d < N_DEV - 1:
                quantize(d + 1)

        xbf[:, pl.ds(my * k_shard, k_shard)] = (
            x_ref[pl.ds(my * m_per, m_per), :] * INV_SCALE
        ).astype(jnp.bfloat16)

        def dequant(d):
            src = lax.rem(my + d, N_DEV)
            recv = pltpu.make_async_remote_copy(
                src_ref=q_comm.at[src],
                dst_ref=q_comm.at[src],
                send_sem=send_sems.at[d - 1],
                recv_sem=recv_sems.at[src],
                device_id=(src,),
                device_id_type=pl.DeviceIdType.MESH,
            )
            recv.wait_recv()
            xbf[:, pl.ds(src * k_shard, k_shard)] = q_comm[src].astype(
                jnp.bfloat16
            )

        dequant(1)
        dequant(3)
        dequant(2)

        for cp in w_cps:
            cp.wait()
        out_ref[...] = (
            jnp.maximum(
                jnp.dot(xbf[...], w_full[...],
                        preferred_element_type=jnp.float32),
                0.0,
            )
            * DEQ_SCALE
        )

        for rdma in sends:
            rdma.wait_send()

    return pl.pallas_call(
        body,
        out_shape=jax.ShapeDtypeStruct((m_per, n), jnp.float32),
        in_specs=[
            pl.BlockSpec(memory_space=pltpu.VMEM),
            pl.BlockSpec(memory_space=pl.ANY),
        ],
        out_specs=pl.BlockSpec(memory_space=pltpu.VMEM),
        scratch_shapes=[
            pltpu.VMEM((N_DEV, m_per, k_shard), jnp.int8),
            pltpu.VMEM((N_DEV, m_per, k_shard), jnp.int8),
            pltpu.VMEM((m_per, k_total), jnp.bfloat16),
            pltpu.VMEM((k_total, n), jnp.float32),
            pltpu.SemaphoreType.DMA((N_DEV - 1,)),
            pltpu.SemaphoreType.DMA((N_DEV,)),
            pltpu.SemaphoreType.DMA((2,)),
        ],
        compiler_params=pltpu.CompilerParams(collective_id=0),
    )(x, w_mat)


# device time: 19320 ns/iter; 1.0296x vs baseline; 1.0296x over previous
import jax
import jax.numpy as jnp
from jax import lax
from jax.experimental import pallas as pl
from jax.experimental.pallas import tpu as pltpu

N_DEV = 4

QCLIP = 5.0
INV_SCALE = 127.0 / QCLIP
DEQ_SCALE = QCLIP / 127.0

_ORDER = (0, 1, 3, 2)


def kernel(x, w_mat):
    m_total, k_shard = x.shape
    k_total, n = w_mat.shape
    m_per = m_total // N_DEV

    def body(x_ref, w_hbm, out_ref,
             q_stage, q_comm, xbf, w_full,
             send_sems, recv_sems, w_sems):
        my = lax.axis_index("i")

        w_cps = []
        for idx, d in enumerate(_ORDER):
            j = lax.rem(my + d, N_DEV)
            cp = pltpu.make_async_copy(
                w_hbm.at[pl.ds(j * k_shard, k_shard), :],
                w_full.at[pl.ds(j * k_shard, k_shard), :],
                w_sems.at[idx],
            )
            cp.start()
            w_cps.append(cp)

        barrier_sem = pltpu.get_barrier_semaphore()
        for d in range(1, N_DEV):
            peer = lax.rem(my + d, N_DEV)
            pl.semaphore_signal(
                barrier_sem, inc=1,
                device_id=(peer,), device_id_type=pl.DeviceIdType.MESH,
            )

        def quantize(d):
            peer = lax.rem(my + d, N_DEV)
            blk = x_ref[pl.ds(peer * m_per, m_per), :]
            q_stage[d] = jnp.clip(
                jnp.round(blk * INV_SCALE), -127.0, 127.0
            ).astype(jnp.int8)

        quantize(1)
        pl.semaphore_wait(barrier_sem, N_DEV - 1)

        sends = []
        for d in range(1, N_DEV):
            peer = lax.rem(my + d, N_DEV)
            rdma = pltpu.make_async_remote_copy(
                src_ref=q_stage.at[d],
                dst_ref=q_comm.at[my],
                send_sem=send_sems.at[d - 1],
                recv_sem=recv_sems.at[my],
                device_id=(peer,),
                device_id_type=pl.DeviceIdType.MESH,
            )
            rdma.start()
            sends.append(rdma)
            if d < N_DEV - 1:
                quantize(d + 1)

        xbf[:, pl.ds(my * k_shard, k_shard)] = (
            x_ref[pl.ds(my * m_per, m_per), :] * INV_SCALE
        ).astype(jnp.bfloat16)

        def dequant(d):
            src = lax.rem(my + d, N_DEV)
            recv = pltpu.make_async_remote_copy(
                src_ref=q_comm.at[src],
                dst_ref=q_comm.at[src],
                send_sem=send_sems.at[d - 1],
                recv_sem=recv_sems.at[src],
                device_id=(src,),
                device_id_type=pl.DeviceIdType.MESH,
            )
            recv.wait_recv()
            xbf[:, pl.ds(src * k_shard, k_shard)] = q_comm[src].astype(
                jnp.bfloat16
            )
            return src

        for idx, d in enumerate(_ORDER):
            src = lax.rem(my + d, N_DEV) if d == 0 else dequant(d)
            w_cps[idx].wait()
            part = jnp.dot(
                xbf[:, pl.ds(src * k_shard, k_shard)],
                w_full[pl.ds(src * k_shard, k_shard), :],
                preferred_element_type=jnp.float32,
            )
            if idx == 0:
                out_ref[...] = part
            elif idx < N_DEV - 1:
                out_ref[...] += part
            else:
                out_ref[...] = (
                    jnp.maximum(out_ref[...] + part, 0.0) * DEQ_SCALE
                )

        for rdma in sends:
            rdma.wait_send()

    return pl.pallas_call(
        body,
        out_shape=jax.ShapeDtypeStruct((m_per, n), jnp.float32),
        in_specs=[
            pl.BlockSpec(memory_space=pltpu.VMEM),
            pl.BlockSpec(memory_space=pl.ANY),
        ],
        out_specs=pl.BlockSpec(memory_space=pltpu.VMEM),
        scratch_shapes=[
            pltpu.VMEM((N_DEV, m_per, k_shard), jnp.int8),
            pltpu.VMEM((N_DEV, m_per, k_shard), jnp.int8),
            pltpu.VMEM((m_per, k_total), jnp.bfloat16),
            pltpu.VMEM((k_total, n), jnp.float32),
            pltpu.SemaphoreType.DMA((N_DEV - 1,)),
            pltpu.SemaphoreType.DMA((N_DEV,)),
            pltpu.SemaphoreType.DMA((N_DEV,)),
        ],
        compiler_params=pltpu.CompilerParams(collective_id=0),
    )(x, w_mat)


# device time: 16758 ns/iter; 1.1870x vs baseline; 1.1529x over previous
import jax
import jax.numpy as jnp
from jax import lax
from jax.experimental import pallas as pl
from jax.experimental.pallas import tpu as pltpu

N_DEV = 4

QCLIP = 5.0
INV_SCALE = 127.0 / QCLIP
DEQ_SCALE = QCLIP / 127.0

_ORDER = (0, 3, 2, 1)


def kernel(x, w_mat):
    m_total, k_shard = x.shape
    k_total, n = w_mat.shape
    m_per = m_total // N_DEV

    def body(x_hbm, w_hbm, out_ref,
             xv, q_stage, q_comm, xbf, w_full,
             send_sems, recv_sems, w_sems, x_sems):
        my = lax.axis_index("i")

        x_cps = {}
        for idx, d in enumerate((1, 2, 3, 0)):
            j = lax.rem(my + d, N_DEV)
            cp = pltpu.make_async_copy(
                x_hbm.at[pl.ds(j * m_per, m_per), :],
                xv.at[pl.ds(j * m_per, m_per), :],
                x_sems.at[idx],
            )
            cp.start()
            x_cps[d] = cp

        def w_fetch(idx):
            j = lax.rem(my + _ORDER[idx], N_DEV)
            cp = pltpu.make_async_copy(
                w_hbm.at[pl.ds(j * k_shard, k_shard), :],
                w_full.at[pl.ds(j * k_shard, k_shard), :],
                w_sems.at[idx],
            )
            cp.start()
            return cp

        w_cps = [w_fetch(0), w_fetch(1)]

        barrier_sem = pltpu.get_barrier_semaphore()
        for d in range(1, N_DEV):
            peer = lax.rem(my + d, N_DEV)
            pl.semaphore_signal(
                barrier_sem, inc=1,
                device_id=(peer,), device_id_type=pl.DeviceIdType.MESH,
            )

        def quantize(d):
            peer = lax.rem(my + d, N_DEV)
            x_cps[d].wait()
            blk = xv[pl.ds(peer * m_per, m_per), :]
            q_stage[d] = jnp.clip(
                jnp.round(blk * INV_SCALE), -127.0, 127.0
            ).astype(jnp.int8)

        quantize(1)
        pl.semaphore_wait(barrier_sem, N_DEV - 1)

        sends = []
        for d in range(1, N_DEV):
            peer = lax.rem(my + d, N_DEV)
            rdma = pltpu.make_async_remote_copy(
                src_ref=q_stage.at[d],
                dst_ref=q_comm.at[my],
                send_sem=send_sems.at[d - 1],
                recv_sem=recv_sems.at[my],
                device_id=(peer,),
                device_id_type=pl.DeviceIdType.MESH,
            )
            rdma.start()
            sends.append(rdma)
            if d < N_DEV - 1:
                quantize(d + 1)

        x_cps[0].wait()
        xbf[:, pl.ds(my * k_shard, k_shard)] = (
            xv[pl.ds(my * m_per, m_per), :] * INV_SCALE
        ).astype(jnp.bfloat16)

        def dequant(d):
            src = lax.rem(my + d, N_DEV)
            recv = pltpu.make_async_remote_copy(
                src_ref=q_comm.at[src],
                dst_ref=q_comm.at[src],
                send_sem=send_sems.at[d - 1],
                recv_sem=recv_sems.at[src],
                device_id=(src,),
                device_id_type=pl.DeviceIdType.MESH,
            )
            recv.wait_recv()
            xbf[:, pl.ds(src * k_shard, k_shard)] = q_comm[src].astype(
                jnp.bfloat16
            )
            return src

        for idx, d in enumerate(_ORDER):
            src = lax.rem(my + d, N_DEV) if d == 0 else dequant(d)
            w_cps[idx].wait()
            if idx + 2 < N_DEV:
                w_cps.append(w_fetch(idx + 2))
            part = jnp.dot(
                xbf[:, pl.ds(src * k_shard, k_shard)],
                w_full[pl.ds(src * k_shard, k_shard), :],
                preferred_element_type=jnp.float32,
            )
            if idx == 0:
                out_ref[...] = part
            elif idx < N_DEV - 1:
                out_ref[...] += part
            else:
                out_ref[...] = (
                    jnp.maximum(out_ref[...] + part, 0.0) * DEQ_SCALE
                )

        for rdma in sends:
            rdma.wait_send()

    return pl.pallas_call(
        body,
        out_shape=jax.ShapeDtypeStruct((m_per, n), jnp.float32),
        in_specs=[
            pl.BlockSpec(memory_space=pl.ANY),
            pl.BlockSpec(memory_space=pl.ANY),
        ],
        out_specs=pl.BlockSpec(memory_space=pltpu.VMEM),
        scratch_shapes=[
            pltpu.VMEM((m_total, k_shard), jnp.float32),
            pltpu.VMEM((N_DEV, m_per, k_shard), jnp.int8),
            pltpu.VMEM((N_DEV, m_per, k_shard), jnp.int8),
            pltpu.VMEM((m_per, k_total), jnp.bfloat16),
            pltpu.VMEM((k_total, n), jnp.float32),
            pltpu.SemaphoreType.DMA((N_DEV - 1,)),
            pltpu.SemaphoreType.DMA((N_DEV,)),
            pltpu.SemaphoreType.DMA((N_DEV,)),
            pltpu.SemaphoreType.DMA((N_DEV,)),
        ],
        compiler_params=pltpu.CompilerParams(collective_id=0),
    )(x, w_mat)
